# speedup vs baseline: 1.4376x; 1.4376x over previous
import functools
import jax
import jax.numpy as jnp
from jax import lax
from jax.experimental import pallas as pl
from jax.experimental.pallas import tpu as pltpu

_E, _TOPK, _H, _I = 16, 2, 1024, 512
_NORM_MIN = 1e-12
_TB = 256


def _router_kernel(x_ref, gwt_ref, logits_ref, combine_ref):
    x = x_ref[...]
    logits = jnp.dot(x, gwt_ref[...], preferred_element_type=jnp.float32)
    logits_ref[...] = logits
    m = jnp.max(logits, axis=-1, keepdims=True)
    ex = jnp.exp(logits - m)
    probs = ex / jnp.sum(ex, axis=-1, keepdims=True)
    iota = lax.broadcasted_iota(jnp.int32, probs.shape, 1)
    m0 = jnp.max(probs, axis=-1, keepdims=True)
    e0 = jnp.min(jnp.where(probs == m0, iota, _E), axis=-1, keepdims=True)
    probs1 = jnp.where(iota == e0, -1.0, probs)
    m1 = jnp.max(probs1, axis=-1, keepdims=True)
    e1 = jnp.min(jnp.where(probs1 == m1, iota, _E), axis=-1, keepdims=True)
    s = jnp.maximum(m0 + m1, _NORM_MIN)
    w0 = m0 / s
    w1 = m1 / s
    combine_ref[...] = jnp.where(iota == e0, w0, 0.0) + jnp.where(iota == e1, w1, 0.0)


def _dense_moe_kernel(x_ref, comb_ref, gw_ref, uw_ref, dw_ref, out_ref):
    e = pl.program_id(1)
    x = x_ref[...]
    g = jnp.dot(x, gw_ref[0], preferred_element_type=jnp.float32)
    u = jnp.dot(x, uw_ref[0], preferred_element_type=jnp.float32)
    h = g * jax.nn.sigmoid(g) * u
    y = jnp.dot(h, dw_ref[0], preferred_element_type=jnp.float32)
    iota = lax.broadcasted_iota(jnp.int32, comb_ref.shape, 1)
    col = jnp.sum(jnp.where(iota == e, comb_ref[...], 0.0), axis=-1, keepdims=True)

    @pl.when(e == 0)
    def _():
        out_ref[...] = jnp.zeros_like(out_ref)

    out_ref[...] += y * col


def _shared_kernel(x_ref, moe_ref, sg_ref, su_ref, sd_ref, out_ref):
    x = x_ref[...]
    g = jnp.dot(x, sg_ref[...], preferred_element_type=jnp.float32)
    u = jnp.dot(x, su_ref[...], preferred_element_type=jnp.float32)
    h = g * jax.nn.sigmoid(g) * u
    out_ref[...] = moe_ref[...] + jnp.dot(h, sd_ref[...], preferred_element_type=jnp.float32)


def kernel(hidden_states, gate_w, expert_gate_w, expert_up_w, expert_down_w,
           shared_gate_w, shared_up_w, shared_down_w):
    b, s, hd = hidden_states.shape
    x = hidden_states.reshape(-1, hd)
    T = x.shape[0]
    nb = T // _TB

    logits, combine = pl.pallas_call(
        _router_kernel,
        grid=(nb,),
        in_specs=[
            pl.BlockSpec((_TB, _H), lambda i: (i, 0)),
            pl.BlockSpec((_H, _E), lambda i: (0, 0)),
        ],
        out_specs=[
            pl.BlockSpec((_TB, _E), lambda i: (i, 0)),
            pl.BlockSpec((_TB, _E), lambda i: (i, 0)),
        ],
        out_shape=[
            jax.ShapeDtypeStruct((T, _E), jnp.float32),
            jax.ShapeDtypeStruct((T, _E), jnp.float32),
        ],
    )(x, gate_w.T)

    moe_out = pl.pallas_call(
        _dense_moe_kernel,
        grid=(nb, _E),
        in_specs=[
            pl.BlockSpec((_TB, _H), lambda i, e: (i, 0)),
            pl.BlockSpec((_TB, _E), lambda i, e: (i, 0)),
            pl.BlockSpec((1, _H, _I), lambda i, e: (e, 0, 0)),
            pl.BlockSpec((1, _H, _I), lambda i, e: (e, 0, 0)),
            pl.BlockSpec((1, _I, _H), lambda i, e: (e, 0, 0)),
        ],
        out_specs=pl.BlockSpec((_TB, _H), lambda i, e: (i, 0)),
        out_shape=jax.ShapeDtypeStruct((T, _H), jnp.float32),
    )(x, combine, expert_gate_w, expert_up_w, expert_down_w)

    out = pl.pallas_call(
        _shared_kernel,
        grid=(nb,),
        in_specs=[
            pl.BlockSpec((_TB, _H), lambda i: (i, 0)),
            pl.BlockSpec((_TB, _H), lambda i: (i, 0)),
            pl.BlockSpec((_H, _I), lambda i: (0, 0)),
            pl.BlockSpec((_H, _I), lambda i: (0, 0)),
            pl.BlockSpec((_I, _H), lambda i: (0, 0)),
        ],
        out_specs=pl.BlockSpec((_TB, _H), lambda i: (i, 0)),
        out_shape=jax.ShapeDtypeStruct((T, _H), jnp.float32),
    )(x, moe_out, shared_gate_w, shared_up_w, shared_down_w)

    return out.reshape(b, s, hd), logits
